# baseline (speedup 1.0000x reference)
import functools

import jax
import jax.numpy as jnp
from jax import lax
from jax.experimental import pallas as pl
from jax.experimental.pallas import tpu as pltpu
from jax.experimental.pallas import tpu_sc as plsc

N = 10000
E = 320000
R = 500
D = 128

NC = 2
NS = 16
NW = NC * NS

CHUNK = 80
TOT_CHUNKS = E // CHUNK

NP = 10240
RPT = NP // NS

BN = 2000


def _pre_body(kg_ref, wn_ref, b_ref, wm_ref, rel_ref, x_ref, y_ref, rm_ref):
    wm = wm_ref[...].astype(jnp.bfloat16)
    x = jnp.dot(kg_ref[...].astype(jnp.bfloat16),
                wn_ref[...].astype(jnp.bfloat16),
                preferred_element_type=jnp.float32)
    x = x + b_ref[...]
    x_ref[...] = x
    y_ref[...] = jnp.dot(x.astype(jnp.bfloat16), wm,
                         preferred_element_type=jnp.float32)

    @pl.when(pl.program_id(0) == 0)
    def _():
        rm_ref[...] = jnp.dot(rel_ref[...].astype(jnp.bfloat16), wm,
                              preferred_element_type=jnp.float32)


def _pre(kg, W_name, b2d, W_msg, rel_emb):
    return pl.pallas_call(
        _pre_body,
        grid=(N // BN,),
        in_specs=[
            pl.BlockSpec((BN, kg.shape[1]), lambda i: (i, 0)),
            pl.BlockSpec(W_name.shape, lambda i: (0, 0)),
            pl.BlockSpec(b2d.shape, lambda i: (0, 0)),
            pl.BlockSpec(W_msg.shape, lambda i: (0, 0)),
            pl.BlockSpec(rel_emb.shape, lambda i: (0, 0)),
        ],
        out_specs=[
            pl.BlockSpec((BN, D), lambda i: (i, 0)),
            pl.BlockSpec((BN, D), lambda i: (i, 0)),
            pl.BlockSpec((R, D), lambda i: (0, 0)),
        ],
        out_shape=[
            jax.ShapeDtypeStruct((N, D), jnp.float32),
            jax.ShapeDtypeStruct((N, D), jnp.float32),
            jax.ShapeDtypeStruct((R, D), jnp.float32),
        ],
    )(kg, W_name, b2d, W_msg, rel_emb)


G = 16
TOT_GROUPS = TOT_CHUNKS // G
FULL_W = TOT_GROUPS - 7 * NW // 8 * 8
GROUPS_LO = TOT_GROUPS // NW


def _sc_agg(y, rel_msg, src1, et1, dst2, z2, z1, o1):
    mesh = plsc.VectorSubcoreMesh(core_axis_name="c", subcore_axis_name="s")

    @functools.partial(
        pl.kernel,
        out_type=(
            jax.ShapeDtypeStruct((NC, NP, D), jnp.float32),
            jax.ShapeDtypeStruct((NC, NP), jnp.float32),
        ),
        mesh=mesh,
        scratch_types=[
            pltpu.VMEM_SHARED((NP, D), jnp.float32),
            pltpu.VMEM_SHARED((NP,), jnp.float32),
            pltpu.VMEM((G * CHUNK,), jnp.int32),
            pltpu.VMEM((G * CHUNK,), jnp.int32),
            pltpu.VMEM((G, CHUNK), jnp.int32),
            pltpu.VMEM((CHUNK, D), jnp.float32),
            pltpu.VMEM((CHUNK, D), jnp.float32),
            pltpu.VMEM((CHUNK, D), jnp.float32),
            pltpu.VMEM((CHUNK, D), jnp.float32),
            pltpu.VMEM((CHUNK,), jnp.float32),
            pltpu.SemaphoreType.DMA,
            pltpu.SemaphoreType.DMA,
            pltpu.SemaphoreType.DMA,
            pltpu.SemaphoreType.DMA,
            pltpu.SemaphoreType.DMA,
        ],
    )
    def body(y_hbm, rm_hbm, src_hbm, et_hbm, dst_hbm, z2_hbm, z1_hbm, o1_hbm,
             agg_hbm, deg_hbm,
             agg_sp, deg_sp, ia_v, ib_v, db_v, ybuf0, rbuf0, ybuf1, rbuf1,
             ones_v, sg0, sg1, ss0, ss1, sd):
        c = lax.axis_index("c")
        s = lax.axis_index("s")
        w = c * NS + s

        row0 = s * RPT
        pltpu.sync_copy(z2_hbm, agg_sp.at[pl.ds(row0, RPT)])
        pltpu.sync_copy(z1_hbm, deg_sp.at[pl.ds(row0, RPT)])

        pltpu.sync_copy(o1_hbm, ones_v)

        ngroups_w = jnp.where(w < FULL_W, GROUPS_LO + 1, GROUPS_LO)
        pairs = ((ybuf0, rbuf0, sg0, ss0), (ybuf1, rbuf1, sg1, ss1))

        def issue_gathers(j, p):
            yb, rb, sg, _ = pairs[p]
            pltpu.async_copy(y_hbm.at[ia_v.at[pl.ds(j * CHUNK, CHUNK)]], yb,
                             sg)
            pltpu.async_copy(rm_hbm.at[ib_v.at[pl.ds(j * CHUNK, CHUNK)]], rb,
                             sg)

        def wait_gathers(j, p):
            yb, rb, sg, _ = pairs[p]
            pltpu.make_async_copy(y_hbm.at[ia_v.at[pl.ds(j * CHUNK, CHUNK)]],
                                  yb, sg).wait()
            pltpu.make_async_copy(rm_hbm.at[ib_v.at[pl.ds(j * CHUNK, CHUNK)]],
                                  rb, sg).wait()

        def issue_scatters(j, p):
            yb, rb, _, ss = pairs[p]
            pltpu.async_copy(yb, agg_sp.at[db_v.at[j]], ss, add=True)
            pltpu.async_copy(rb, agg_sp.at[db_v.at[j]], ss, add=True)
            pltpu.async_copy(ones_v, deg_sp.at[db_v.at[j]], sd, add=True)

        def wait_scatters(j, p):
            yb, rb, _, ss = pairs[p]
            pltpu.make_async_copy(yb, agg_sp.at[db_v.at[j]], ss).wait()
            pltpu.make_async_copy(rb, agg_sp.at[db_v.at[j]], ss).wait()
            pltpu.make_async_copy(ones_v, deg_sp.at[db_v.at[j]], sd).wait()

        @pl.loop(0, ngroups_w)
        def _(grp):
            base = (w + NW * grp) * G
            pltpu.sync_copy(src_hbm.at[pl.ds(base * CHUNK, G * CHUNK)], ia_v)
            pltpu.sync_copy(et_hbm.at[pl.ds(base * CHUNK, G * CHUNK)], ib_v)
            pltpu.sync_copy(dst_hbm.at[pl.ds(base, G)], db_v)

            issue_gathers(0, 0)
            wait_gathers(0, 0)
            issue_scatters(0, 0)
            issue_gathers(1, 1)

            @pl.loop(1, G - 1, step=2)
            def _(k):
                wait_gathers(k, 1)
                issue_scatters(k, 1)
                wait_scatters(k, 0)
                issue_gathers(k + 1, 0)
                wait_gathers(k + 1, 0)
                issue_scatters(k + 1, 0)
                wait_scatters(k + 1, 1)
                issue_gathers(k + 2, 1)

            wait_gathers(G - 1, 1)
            issue_scatters(G - 1, 1)
            wait_scatters(G - 2, 0)
            wait_scatters(G - 1, 1)

        plsc.subcore_barrier()
        pltpu.sync_copy(agg_sp.at[pl.ds(row0, RPT)],
                        agg_hbm.at[c, pl.ds(row0, RPT)])
        pltpu.sync_copy(deg_sp.at[pl.ds(row0, RPT)],
                        deg_hbm.at[c, pl.ds(row0, RPT)])

    return body(y, rel_msg, src1, et1, dst2, z2, z1, o1)


def _mix_body(x_ref, a_ref, d_ref, sw_ref, out_ref):
    alpha = jax.nn.sigmoid(sw_ref[0, 0])
    agg = a_ref[0, :, :] + a_ref[1, :, :]
    deg = d_ref[0, :, :] + d_ref[1, :, :]
    agg = agg / jnp.maximum(deg, 1.0)
    out_ref[...] = alpha * x_ref[...] + (1.0 - alpha) * jnp.maximum(agg, 0.0)


def _mix(x, agg, deg3, sw2d):
    return pl.pallas_call(
        _mix_body,
        grid=(N // BN,),
        in_specs=[
            pl.BlockSpec((BN, D), lambda i: (i, 0)),
            pl.BlockSpec((NC, BN, D), lambda i: (0, i, 0)),
            pl.BlockSpec((NC, BN, 1), lambda i: (0, i, 0)),
            pl.BlockSpec((1, 1), lambda i: (0, 0)),
        ],
        out_specs=pl.BlockSpec((BN, D), lambda i: (i, 0)),
        out_shape=jax.ShapeDtypeStruct((N, D), jnp.float32),
    )(x, agg, deg3, sw2d)


def kernel(kg_name_embed, edge_index, edge_type, W_name, b_name, rel_emb,
           W_msg, skip_w):
    src1 = edge_index[0].astype(jnp.int32)
    et1 = edge_type.astype(jnp.int32)
    dst2 = edge_index[1].astype(jnp.int32).reshape(TOT_CHUNKS, CHUNK)

    z2 = jnp.zeros((RPT, D), jnp.float32)
    z1 = jnp.zeros((RPT,), jnp.float32)
    o1 = jnp.ones((CHUNK,), jnp.float32)

    x, y, rel_msg = _pre(kg_name_embed, W_name, b_name.reshape(1, D), W_msg,
                         rel_emb)
    agg, deg = _sc_agg(y, rel_msg, src1, et1, dst2, z2, z1, o1)
    out = _mix(x, agg, deg.reshape(NC, NP, 1), skip_w.reshape(1, 1))
    return out


# speedup vs baseline: 1.0349x; 1.0349x over previous
import functools

import jax
import jax.numpy as jnp
from jax import lax
from jax.experimental import pallas as pl
from jax.experimental.pallas import tpu as pltpu
from jax.experimental.pallas import tpu_sc as plsc

N = 10000
E = 320000
R = 500
D = 128

NC = 2
NS = 16
NW = NC * NS

CHUNK = 80
TOT_CHUNKS = E // CHUNK

NP = 10240
RPT = NP // NS

BN = 2000


def _pre_body(kg_ref, wn_ref, b_ref, wm_ref, rel_ref, x_ref, y_ref, rm_ref):
    wm = wm_ref[...].astype(jnp.bfloat16)
    x = jnp.dot(kg_ref[...].astype(jnp.bfloat16),
                wn_ref[...].astype(jnp.bfloat16),
                preferred_element_type=jnp.float32)
    x = x + b_ref[...]
    x_ref[...] = x
    y_ref[...] = jnp.dot(x.astype(jnp.bfloat16), wm,
                         preferred_element_type=jnp.float32)

    @pl.when(pl.program_id(0) == 0)
    def _():
        rm_ref[...] = jnp.dot(rel_ref[...].astype(jnp.bfloat16), wm,
                              preferred_element_type=jnp.float32)


def _pre(kg, W_name, b2d, W_msg, rel_emb):
    return pl.pallas_call(
        _pre_body,
        grid=(N // BN,),
        in_specs=[
            pl.BlockSpec((BN, kg.shape[1]), lambda i: (i, 0)),
            pl.BlockSpec(W_name.shape, lambda i: (0, 0)),
            pl.BlockSpec(b2d.shape, lambda i: (0, 0)),
            pl.BlockSpec(W_msg.shape, lambda i: (0, 0)),
            pl.BlockSpec(rel_emb.shape, lambda i: (0, 0)),
        ],
        out_specs=[
            pl.BlockSpec((BN, D), lambda i: (i, 0)),
            pl.BlockSpec((BN, D), lambda i: (i, 0)),
            pl.BlockSpec((R, D), lambda i: (0, 0)),
        ],
        out_shape=[
            jax.ShapeDtypeStruct((N, D), jnp.float32),
            jax.ShapeDtypeStruct((N, D), jnp.float32),
            jax.ShapeDtypeStruct((R, D), jnp.float32),
        ],
    )(kg, W_name, b2d, W_msg, rel_emb)


G = 16
TOT_GROUPS = TOT_CHUNKS // G
FULL_W = TOT_GROUPS - 7 * NW // 8 * 8
GROUPS_LO = TOT_GROUPS // NW


def _sc_agg(y, rel_msg, ei3, et2, z2, z1):
    mesh = plsc.VectorSubcoreMesh(core_axis_name="c", subcore_axis_name="s")

    @functools.partial(
        pl.kernel,
        out_type=(
            jax.ShapeDtypeStruct((NC, NP, D), jnp.float32),
            jax.ShapeDtypeStruct((NC, NP), jnp.float32),
        ),
        mesh=mesh,
        scratch_types=[
            pltpu.VMEM_SHARED((NP, D), jnp.float32),
            pltpu.VMEM_SHARED((NP,), jnp.float32),
            pltpu.VMEM((G, CHUNK), jnp.int32),
            pltpu.VMEM((G, CHUNK), jnp.int32),
            pltpu.VMEM((G, CHUNK), jnp.int32),
            pltpu.VMEM((CHUNK, D), jnp.float32),
            pltpu.VMEM((CHUNK, D), jnp.float32),
            pltpu.VMEM((CHUNK, D), jnp.float32),
            pltpu.VMEM((CHUNK, D), jnp.float32),
            pltpu.VMEM((CHUNK,), jnp.float32),
            pltpu.SemaphoreType.DMA,
            pltpu.SemaphoreType.DMA,
            pltpu.SemaphoreType.DMA,
            pltpu.SemaphoreType.DMA,
            pltpu.SemaphoreType.DMA,
        ],
    )
    def body(y_hbm, rm_hbm, ei_hbm, et_hbm, z2_hbm, z1_hbm,
             agg_hbm, deg_hbm,
             agg_sp, deg_sp, ia_v, ib_v, db_v, ybuf0, rbuf0, ybuf1, rbuf1,
             ones_v, sg0, sg1, ss0, ss1, sd):
        c = lax.axis_index("c")
        s = lax.axis_index("s")
        w = c * NS + s

        row0 = s * RPT
        pltpu.sync_copy(z2_hbm, agg_sp.at[pl.ds(row0, RPT)])
        pltpu.sync_copy(z1_hbm, deg_sp.at[pl.ds(row0, RPT)])

        @pl.loop(0, CHUNK, step=16)
        def _(j):
            ones_v[pl.ds(j, 16)] = jnp.ones((16,), jnp.float32)

        ngroups_w = jnp.where(w < FULL_W, GROUPS_LO + 1, GROUPS_LO)
        pairs = ((ybuf0, rbuf0, sg0, ss0), (ybuf1, rbuf1, sg1, ss1))

        def issue_gathers(j, p):
            yb, rb, sg, _ = pairs[p]
            pltpu.async_copy(y_hbm.at[ia_v.at[j]], yb, sg)
            pltpu.async_copy(rm_hbm.at[ib_v.at[j]], rb, sg)

        def wait_gathers(j, p):
            yb, rb, sg, _ = pairs[p]
            pltpu.make_async_copy(y_hbm.at[ia_v.at[j]], yb, sg).wait()
            pltpu.make_async_copy(rm_hbm.at[ib_v.at[j]], rb, sg).wait()

        def issue_scatters(j, p):
            yb, rb, _, ss = pairs[p]
            pltpu.async_copy(yb, agg_sp.at[db_v.at[j]], ss, add=True)
            pltpu.async_copy(rb, agg_sp.at[db_v.at[j]], ss, add=True)
            pltpu.async_copy(ones_v, deg_sp.at[db_v.at[j]], sd, add=True)

        def wait_scatters(j, p):
            yb, rb, _, ss = pairs[p]
            pltpu.make_async_copy(yb, agg_sp.at[db_v.at[j]], ss).wait()
            pltpu.make_async_copy(rb, agg_sp.at[db_v.at[j]], ss).wait()
            pltpu.make_async_copy(ones_v, deg_sp.at[db_v.at[j]], sd).wait()

        @pl.loop(0, ngroups_w)
        def _(grp):
            base = (w + NW * grp) * G
            pltpu.sync_copy(ei_hbm.at[0, pl.ds(base, G)], ia_v)
            pltpu.sync_copy(et_hbm.at[pl.ds(base, G)], ib_v)
            pltpu.sync_copy(ei_hbm.at[1, pl.ds(base, G)], db_v)

            issue_gathers(0, 0)
            wait_gathers(0, 0)
            issue_scatters(0, 0)
            issue_gathers(1, 1)

            @pl.loop(1, G - 1, step=2)
            def _(k):
                wait_gathers(k, 1)
                issue_scatters(k, 1)
                wait_scatters(k, 0)
                issue_gathers(k + 1, 0)
                wait_gathers(k + 1, 0)
                issue_scatters(k + 1, 0)
                wait_scatters(k + 1, 1)
                issue_gathers(k + 2, 1)

            wait_gathers(G - 1, 1)
            issue_scatters(G - 1, 1)
            wait_scatters(G - 2, 0)
            wait_scatters(G - 1, 1)

        plsc.subcore_barrier()
        pltpu.sync_copy(agg_sp.at[pl.ds(row0, RPT)],
                        agg_hbm.at[c, pl.ds(row0, RPT)])
        pltpu.sync_copy(deg_sp.at[pl.ds(row0, RPT)],
                        deg_hbm.at[c, pl.ds(row0, RPT)])

    return body(y, rel_msg, ei3, et2, z2, z1)


def _mix_body(x_ref, a_ref, d_ref, sw_ref, out_ref):
    alpha = jax.nn.sigmoid(sw_ref[0, 0])
    agg = a_ref[0, :, :] + a_ref[1, :, :]
    deg = d_ref[0, :, :] + d_ref[1, :, :]
    agg = agg / jnp.maximum(deg, 1.0)
    out_ref[...] = alpha * x_ref[...] + (1.0 - alpha) * jnp.maximum(agg, 0.0)


def _mix(x, agg, deg3, sw2d):
    return pl.pallas_call(
        _mix_body,
        grid=(N // BN,),
        in_specs=[
            pl.BlockSpec((BN, D), lambda i: (i, 0)),
            pl.BlockSpec((NC, BN, D), lambda i: (0, i, 0)),
            pl.BlockSpec((NC, BN, 1), lambda i: (0, i, 0)),
            pl.BlockSpec((1, 1), lambda i: (0, 0)),
        ],
        out_specs=pl.BlockSpec((BN, D), lambda i: (i, 0)),
        out_shape=jax.ShapeDtypeStruct((N, D), jnp.float32),
    )(x, agg, deg3, sw2d)


def kernel(kg_name_embed, edge_index, edge_type, W_name, b_name, rel_emb,
           W_msg, skip_w):
    ei3 = edge_index.astype(jnp.int32).reshape(NC, TOT_CHUNKS, CHUNK)
    et2 = edge_type.astype(jnp.int32).reshape(TOT_CHUNKS, CHUNK)

    z2 = jnp.zeros((RPT, D), jnp.float32)
    z1 = jnp.zeros((RPT,), jnp.float32)

    x, y, rel_msg = _pre(kg_name_embed, W_name, b_name.reshape(1, D), W_msg,
                         rel_emb)
    agg, deg = _sc_agg(y, rel_msg, ei3, et2, z2, z1)
    out = _mix(x, agg, deg.reshape(NC, NP, 1), skip_w.reshape(1, 1))
    return out
